# Initial kernel scaffold; baseline (speedup 1.0000x reference)
import jax
import jax.numpy as jnp
from jax.experimental import pallas as pl


def kernel(input, weights):
    raise NotImplementedError("write your pallas kernel here")


# speedup vs baseline: 1.2282x; 1.2282x over previous
import functools

import jax
import jax.numpy as jnp
from jax import lax
from jax.experimental import pallas as pl
from jax.experimental.pallas import tpu as pltpu
from jax.experimental.pallas import tpu_sc as plsc

_POS_OFFSET = 2


def kernel(input, weights):
    b, t = input.shape
    d = weights.shape[1]

    NC, NS = 2, 16
    NW = NC * NS
    rows_per_w = t // NW
    CHUNK = 128
    n_chunks = rows_per_w // CHUNK

    mesh = plsc.VectorSubcoreMesh(core_axis_name="c", subcore_axis_name="s")

    @functools.partial(
        pl.kernel,
        mesh=mesh,
        out_type=jax.ShapeDtypeStruct((b * t * d,), weights.dtype),
        scratch_types=[
            pltpu.VMEM((CHUNK * d,), weights.dtype),
            pltpu.SemaphoreType.DMA,
        ],
    )
    def _posemb(w_hbm, out_hbm, buf, sem):
        wid = lax.axis_index("s") * NC + lax.axis_index("c")
        base = wid * rows_per_w
        for ci in range(n_chunks):
            r0 = base + ci * CHUNK
            pltpu.sync_copy(w_hbm.at[pl.ds((_POS_OFFSET + r0) * d, CHUNK * d)], buf)
            copies = [
                pltpu.async_copy(buf, out_hbm.at[pl.ds((bi * t + r0) * d, CHUNK * d)], sem)
                for bi in range(b)
            ]
            for cp in copies:
                cp.wait()

    return _posemb(weights.reshape(-1)).reshape(b, t, d)
